# Initial kernel scaffold; baseline (speedup 1.0000x reference)
import jax
import jax.numpy as jnp
from jax.experimental import pallas as pl


def kernel(x, edge_index, batch_data, W1, b1, W2, b2, W3, b3):
    raise NotImplementedError("write your pallas kernel here")


# speedup vs baseline: 10.6998x; 10.6998x over previous
import functools
import jax
import jax.numpy as jnp
from jax import lax
from jax.experimental import pallas as pl
from jax.experimental.pallas import tpu as pltpu
from jax.experimental.pallas import tpu_sc as plsc

_N = 10000
_E = 320000
_G = 64
_D = 128
_NC = 2
_NS = 16
_NW = _NC * _NS
_CHUNK = 128
_CH = -(-_E // (_NW * _CHUNK))
_EPW = _CH * _CHUNK
_EPAD = _EPW * _NW
_NPAD = 10240
_RPT = _NPAD // _NS
_DDEG = 16
_BLK = 1000


def _deg_body(dst_hbm, out_hbm, idx_d, ones, zbuf, acc):
    c = lax.axis_index("c")
    s = lax.axis_index("s")
    wid = c * _NS + s
    pltpu.sync_copy(dst_hbm.at[wid], idx_d)
    for r in range(_CHUNK):
        ones[r] = jnp.ones((16,), jnp.float32)
    for r in range(16):
        zbuf[r] = jnp.zeros((16,), jnp.float32)
    base = s * _RPT

    def zloop(k, carry):
        pltpu.sync_copy(zbuf, acc.at[pl.ds(base + k * 16, 16)])
        return carry

    lax.fori_loop(0, _RPT // 16, zloop, 0)
    plsc.subcore_barrier()

    def eloop(j, carry):
        pltpu.sync_copy(ones, acc.at[idx_d.at[j]], add=True)
        return carry

    lax.fori_loop(0, _CH, eloop, 0)
    plsc.subcore_barrier()
    pltpu.sync_copy(acc.at[pl.ds(base, _RPT)], out_hbm.at[c, pl.ds(base, _RPT)])


@functools.cache
def _deg_call():
    return pl.kernel(
        _deg_body,
        out_type=jax.ShapeDtypeStruct((_NC, _NPAD, _DDEG), jnp.float32),
        mesh=plsc.VectorSubcoreMesh(core_axis_name="c", subcore_axis_name="s"),
        scratch_types=[
            pltpu.VMEM((_CH, _CHUNK), jnp.int32),
            pltpu.VMEM((_CHUNK, _DDEG), jnp.float32),
            pltpu.VMEM((16, _DDEG), jnp.float32),
            pltpu.VMEM_SHARED((_NPAD, _DDEG), jnp.float32),
        ],
    )


def _agg_body(g_hbm, src_hbm, dst_hbm, out_hbm, idx_s, idx_d, rows, zbuf, acc,
              gsem):
    c = lax.axis_index("c")
    s = lax.axis_index("s")
    wid = c * _NS + s
    pltpu.sync_copy(src_hbm.at[wid], idx_s)
    pltpu.sync_copy(dst_hbm.at[wid], idx_d)
    for r in range(16):
        for q in range(_D // 16):
            zbuf[r, pl.ds(q * 16, 16)] = jnp.zeros((16,), jnp.float32)
    base = s * _RPT

    def zloop(k, carry):
        pltpu.sync_copy(zbuf, acc.at[pl.ds(base + k * 16, 16)])
        return carry

    lax.fori_loop(0, _RPT // 16, zloop, 0)
    plsc.subcore_barrier()

    def eloop(j, carry):
        pltpu.async_copy(g_hbm.at[idx_s.at[j]], rows, gsem).wait()
        pltpu.sync_copy(rows, acc.at[idx_d.at[j]], add=True)
        return carry

    lax.fori_loop(0, _CH, eloop, 0)
    plsc.subcore_barrier()
    pltpu.sync_copy(acc.at[pl.ds(base, _RPT)], out_hbm.at[c, pl.ds(base, _RPT)])


@functools.cache
def _agg_call():
    return pl.kernel(
        _agg_body,
        out_type=jax.ShapeDtypeStruct((_NC, _NPAD, _D), jnp.float32),
        mesh=plsc.VectorSubcoreMesh(core_axis_name="c", subcore_axis_name="s"),
        scratch_types=[
            pltpu.VMEM((_CH, _CHUNK), jnp.int32),
            pltpu.VMEM((_CH, _CHUNK), jnp.int32),
            pltpu.VMEM((_CHUNK, _D), jnp.float32),
            pltpu.VMEM((16, _D), jnp.float32),
            pltpu.VMEM_SHARED((_NPAD, _D), jnp.float32),
            pltpu.SemaphoreType.DMA,
        ],
    )


def _pre_body(x_ref, w_ref, d0_ref, d1_ref, g_ref, dis_ref):
    deg = d0_ref[...] + d1_ref[...] + 1.0
    dis = lax.rsqrt(deg)
    hw = jnp.dot(x_ref[...], w_ref[...], preferred_element_type=jnp.float32)
    g_ref[...] = hw * dis
    dis_ref[...] = dis


def _mid_body(p0_ref, p1_ref, gp_ref, dis_ref, b_ref, w_ref, g_ref):
    dis = dis_ref[...]
    agg = (p0_ref[...] + p1_ref[...] + gp_ref[...]) * dis + b_ref[...]
    h = jnp.maximum(agg, 0.0)
    g_ref[...] = jnp.dot(h, w_ref[...], preferred_element_type=jnp.float32) * dis


def _post_body(p0_ref, p1_ref, gp_ref, dis_ref, b_ref, batch_ref, out_ref):
    agg = (p0_ref[...] + p1_ref[...] + gp_ref[...]) * dis_ref[...] + b_ref[...]
    h = jnp.maximum(agg, 0.0)
    scores = jnp.max(h, axis=1, keepdims=True)
    gid = lax.broadcasted_iota(jnp.int32, (_BLK, _G), 1)
    masked = jnp.where(batch_ref[...] == gid, scores, -jnp.inf)
    part = jnp.max(masked, axis=0, keepdims=True)

    @pl.when(pl.program_id(0) == 0)
    def _():
        out_ref[...] = jnp.full((1, _G), -jnp.inf, jnp.float32)

    out_ref[...] = jnp.maximum(out_ref[...], part)


def _row_spec(width):
    return pl.BlockSpec((_BLK, width), lambda i: (i, 0))


def _full_spec(shape):
    return pl.BlockSpec(shape, lambda i: tuple(0 for _ in shape))


_GRID = _N // _BLK

_pre_call = pl.pallas_call(
    _pre_body,
    grid=(_GRID,),
    in_specs=[_row_spec(8), _full_spec((8, _D)), _row_spec(1), _row_spec(1)],
    out_specs=[_row_spec(_D), _row_spec(1)],
    out_shape=[
        jax.ShapeDtypeStruct((_N, _D), jnp.float32),
        jax.ShapeDtypeStruct((_N, 1), jnp.float32),
    ],
)

_mid_call = pl.pallas_call(
    _mid_body,
    grid=(_GRID,),
    in_specs=[_row_spec(_D), _row_spec(_D), _row_spec(_D), _row_spec(1),
              _full_spec((1, _D)), _full_spec((_D, _D))],
    out_specs=_row_spec(_D),
    out_shape=jax.ShapeDtypeStruct((_N, _D), jnp.float32),
)

_post_call = pl.pallas_call(
    _post_body,
    grid=(_GRID,),
    in_specs=[_row_spec(_D), _row_spec(_D), _row_spec(_D), _row_spec(1),
              _full_spec((1, _D)), _row_spec(1)],
    out_specs=pl.BlockSpec((1, _G), lambda i: (0, 0)),
    out_shape=jax.ShapeDtypeStruct((1, _G), jnp.float32),
)


@jax.jit
def _run(x, edge_index, batch_data, W1, b1, W2, b2, W3, b3):
    src = edge_index[0]
    dst = edge_index[1]
    pad = _EPAD - _E
    src3 = jnp.concatenate([src, jnp.zeros((pad,), jnp.int32)]
                           ).reshape(_NW, _CH, _CHUNK)
    dst3 = jnp.concatenate([dst, jnp.full((pad,), _N, jnp.int32)]
                           ).reshape(_NW, _CH, _CHUNK)

    dparts = _deg_call()(dst3)
    d0 = dparts[0, :_N, :1]
    d1 = dparts[1, :_N, :1]

    xp = jnp.pad(x, ((0, 0), (0, 1)))
    w1p = jnp.pad(W1, ((0, 1), (0, 0)))
    g1, dis = _pre_call(xp, w1p, d0, d1)

    p = _agg_call()(g1, src3, dst3)
    g2 = _mid_call(p[0, :_N], p[1, :_N], g1, dis, b1.reshape(1, _D), W2)
    p = _agg_call()(g2, src3, dst3)
    g3 = _mid_call(p[0, :_N], p[1, :_N], g2, dis, b2.reshape(1, _D), W3)
    p = _agg_call()(g3, src3, dst3)
    pooled = _post_call(p[0, :_N], p[1, :_N], g3, dis, b3.reshape(1, _D),
                        batch_data.reshape(_N, 1))
    return pooled.reshape(_G)


def kernel(x, edge_index, batch_data, W1, b1, W2, b2, W3, b3):
    return _run(x, edge_index, batch_data, W1, b1, W2, b2, W3, b3)
